# speedup vs baseline: 1.1237x; 1.1237x over previous
import functools

import jax
import jax.numpy as jnp
from jax import lax
from jax.experimental import pallas as pl
from jax.experimental.pallas import tpu as pltpu
from jax.experimental.pallas import tpu_sc as plsc

_B, _L, _D = 4, 4096, 1024
_H = _D // 2
_N = _B * _L
_NW = 32
_R = _N // _NW
_C = 32
_K = _R // _C
_LANES = 16


def _body(x_hbm, p0_hbm, p1_hbm, pe1_hbm, pe2_hbm, o_hbm,
          idx0_v, idx1_v, g1_v, g2_v, acc_v, sem_in, sem_g1, sem_g2):
    wid = lax.axis_index("s") * 2 + lax.axis_index("c")

    def chunk(k, carry):
        r = wid * _K + k
        base = r * _C
        pltpu.sync_copy(p0_hbm.at[r], idx0_v)
        pltpu.sync_copy(p1_hbm.at[r], idx1_v)
        cin = pltpu.async_copy(x_hbm.at[pl.ds(base, _C)], acc_v, sem_in)
        cg1 = pltpu.async_copy(pe1_hbm.at[idx0_v], g1_v, sem_g1)
        cg2 = pltpu.async_copy(pe2_hbm.at[idx1_v], g2_v, sem_g2)
        cin.wait()
        cg1.wait()
        cg2.wait()

        def row(i, carry2):
            for j in range(_H // _LANES):
                sl = pl.ds(j * _LANES, _LANES)
                sl2 = pl.ds(_H + j * _LANES, _LANES)
                plsc.addupdate(acc_v.at[i, sl], g1_v[i, sl])
                plsc.addupdate(acc_v.at[i, sl2], g2_v[i, sl])
            return carry2

        lax.fori_loop(0, _C, row, 0)
        pltpu.sync_copy(acc_v, o_hbm.at[pl.ds(base, _C)])
        return carry

    lax.fori_loop(0, _K, chunk, 0)


@jax.jit
def kernel(inputs, positions, pe1, pe2):
    x = inputs.reshape(_N, _D)
    pos = positions.astype(jnp.int32).reshape(_N, 2)
    p0 = pos[:, 0].reshape(_NW * _K, _C)
    p1 = pos[:, 1].reshape(_NW * _K, _C)

    mesh = plsc.VectorSubcoreMesh(core_axis_name="c", subcore_axis_name="s")
    run = functools.partial(
        pl.kernel,
        out_type=jax.ShapeDtypeStruct((_N, _D), jnp.float32),
        mesh=mesh,
        scratch_types=[
            pltpu.VMEM((_C,), jnp.int32),
            pltpu.VMEM((_C,), jnp.int32),
            pltpu.VMEM((_C, _H), jnp.float32),
            pltpu.VMEM((_C, _H), jnp.float32),
            pltpu.VMEM((_C, _D), jnp.float32),
            pltpu.SemaphoreType.DMA,
            pltpu.SemaphoreType.DMA,
            pltpu.SemaphoreType.DMA,
        ],
    )(_body)
    out = run(x, p0, p1, pe1, pe2)
    return out.reshape(_B, _L, _D)


# speedup vs baseline: 1.3839x; 1.2316x over previous
import functools

import jax
import jax.numpy as jnp
from jax import lax
from jax.experimental import pallas as pl
from jax.experimental.pallas import tpu as pltpu
from jax.experimental.pallas import tpu_sc as plsc

_B, _L, _D = 4, 4096, 1024
_H = _D // 2
_N = _B * _L
_NW = 32
_R = _N // _NW
_C = 16
_K = _R // _C
_NBUF = 2
_LANES = 16


def _body(x_hbm, p0_hbm, p1_hbm, pe1_hbm, pe2_hbm, o_hbm,
          idx0_v, idx1_v, g1_v, g2_v, acc_v, sem_in, sem_g1, sem_g2,
          sem_out):
    wid = lax.axis_index("s") * 2 + lax.axis_index("c")
    base0 = wid * _R
    pltpu.sync_copy(p0_hbm.at[wid], idx0_v)
    pltpu.sync_copy(p1_hbm.at[wid], idx1_v)

    def issue_loads(k, b):
        base = base0 + k * _C
        pltpu.async_copy(x_hbm.at[pl.ds(base, _C)], acc_v.at[b], sem_in.at[b])
        pltpu.async_copy(pe1_hbm.at[idx0_v.at[k]], g1_v.at[b], sem_g1.at[b])
        pltpu.async_copy(pe2_hbm.at[idx1_v.at[k]], g2_v.at[b], sem_g2.at[b])

    def wait_loads(k, b):
        base = base0 + k * _C
        pltpu.make_async_copy(x_hbm.at[pl.ds(base, _C)], acc_v.at[b],
                              sem_in.at[b]).wait()
        pltpu.make_async_copy(pe1_hbm.at[idx0_v.at[k]], g1_v.at[b],
                              sem_g1.at[b]).wait()
        pltpu.make_async_copy(pe2_hbm.at[idx1_v.at[k]], g2_v.at[b],
                              sem_g2.at[b]).wait()

    def issue_out(k, b):
        base = base0 + k * _C
        pltpu.async_copy(acc_v.at[b], o_hbm.at[pl.ds(base, _C)], sem_out.at[b])

    def wait_out(k, b):
        base = base0 + k * _C
        pltpu.make_async_copy(acc_v.at[b], o_hbm.at[pl.ds(base, _C)],
                              sem_out.at[b]).wait()

    def compute(b):
        def row(i, carry):
            for j in range(_H // _LANES):
                sl = pl.ds(j * _LANES, _LANES)
                sl2 = pl.ds(_H + j * _LANES, _LANES)
                plsc.addupdate(acc_v.at[b, i, sl], g1_v[b, i, sl])
                plsc.addupdate(acc_v.at[b, i, sl2], g2_v[b, i, sl])
            return carry
        lax.fori_loop(0, _C, row, 0)

    issue_loads(0, 0)
    issue_loads(1, 1)

    def step(t, carry):
        k0 = 2 * t
        wait_loads(k0, 0)
        compute(0)
        issue_out(k0, 0)
        wait_loads(k0 + 1, 1)
        compute(1)
        issue_out(k0 + 1, 1)
        wait_out(k0, 0)
        issue_loads(k0 + 2, 0)
        wait_out(k0 + 1, 1)
        issue_loads(k0 + 3, 1)
        return carry

    lax.fori_loop(0, _K // 2 - 1, step, 0)
    kl = _K - 2
    wait_loads(kl, 0)
    compute(0)
    issue_out(kl, 0)
    wait_loads(kl + 1, 1)
    compute(1)
    issue_out(kl + 1, 1)
    wait_out(kl, 0)
    wait_out(kl + 1, 1)


@jax.jit
def kernel(inputs, positions, pe1, pe2):
    x = inputs.reshape(_N, _D)
    pos = positions.astype(jnp.int32).reshape(_N, 2)
    p0 = pos[:, 0].reshape(_NW, _K, _C)
    p1 = pos[:, 1].reshape(_NW, _K, _C)

    mesh = plsc.VectorSubcoreMesh(core_axis_name="c", subcore_axis_name="s")
    run = functools.partial(
        pl.kernel,
        out_type=jax.ShapeDtypeStruct((_N, _D), jnp.float32),
        mesh=mesh,
        scratch_types=[
            pltpu.VMEM((_K, _C), jnp.int32),
            pltpu.VMEM((_K, _C), jnp.int32),
            pltpu.VMEM((_NBUF, _C, _H), jnp.float32),
            pltpu.VMEM((_NBUF, _C, _H), jnp.float32),
            pltpu.VMEM((_NBUF, _C, _D), jnp.float32),
            pltpu.SemaphoreType.DMA((_NBUF,)),
            pltpu.SemaphoreType.DMA((_NBUF,)),
            pltpu.SemaphoreType.DMA((_NBUF,)),
            pltpu.SemaphoreType.DMA((_NBUF,)),
        ],
    )(_body)
    out = run(x, p0, p1, pe1, pe2)
    return out.reshape(_B, _L, _D)


# speedup vs baseline: 1.9018x; 1.3742x over previous
import functools

import jax
import jax.numpy as jnp
from jax import lax
from jax.experimental import pallas as pl
from jax.experimental.pallas import tpu as pltpu
from jax.experimental.pallas import tpu_sc as plsc

_B, _L, _D = 4, 4096, 1024
_H = _D // 2
_N = _B * _L
_NW = 32
_R = _N // _NW
_C = 16
_K = _R // _C
_NBUF = 2
_LANES = 16


def _body(x_hbm, p0_hbm, p1_hbm, pe1_hbm, pe2_hbm, o_hbm,
          idx0_v, idx1_v, g1_v, g2_v, acc_v, sem_in, sem_g1, sem_g2,
          sem_out):
    wid = lax.axis_index("s") * 2 + lax.axis_index("c")
    base0 = wid * _R
    pltpu.sync_copy(p0_hbm.at[wid], idx0_v)
    pltpu.sync_copy(p1_hbm.at[wid], idx1_v)

    def issue_loads(k, b):
        base = base0 + k * _C
        pltpu.async_copy(x_hbm.at[pl.ds(base, _C)], acc_v.at[b], sem_in.at[b])
        pltpu.async_copy(pe1_hbm.at[idx0_v.at[k]], g1_v.at[b], sem_g1.at[b])
        pltpu.async_copy(pe2_hbm.at[idx1_v.at[k]], g2_v.at[b], sem_g2.at[b])

    def wait_loads(k, b):
        base = base0 + k * _C
        pltpu.make_async_copy(x_hbm.at[pl.ds(base, _C)], acc_v.at[b],
                              sem_in.at[b]).wait()
        pltpu.make_async_copy(pe1_hbm.at[idx0_v.at[k]], g1_v.at[b],
                              sem_g1.at[b]).wait()
        pltpu.make_async_copy(pe2_hbm.at[idx1_v.at[k]], g2_v.at[b],
                              sem_g2.at[b]).wait()

    def issue_out(k, b):
        base = base0 + k * _C
        pltpu.async_copy(acc_v.at[b], o_hbm.at[pl.ds(base, _C)], sem_out.at[b])

    def wait_out(k, b):
        base = base0 + k * _C
        pltpu.make_async_copy(acc_v.at[b], o_hbm.at[pl.ds(base, _C)],
                              sem_out.at[b]).wait()

    def compute(b):
        @plsc.parallel_loop(0, _C, unroll=2)
        def row(i):
            for j in range(_H // _LANES):
                sl = pl.ds(j * _LANES, _LANES)
                sl2 = pl.ds(_H + j * _LANES, _LANES)
                plsc.addupdate(acc_v.at[b, i, sl], g1_v[b, i, sl])
                plsc.addupdate(acc_v.at[b, i, sl2], g2_v[b, i, sl])

    issue_loads(0, 0)
    issue_loads(1, 1)

    def step(t, carry):
        k0 = 2 * t
        wait_loads(k0, 0)
        compute(0)
        issue_out(k0, 0)
        wait_loads(k0 + 1, 1)
        compute(1)
        issue_out(k0 + 1, 1)
        wait_out(k0, 0)
        issue_loads(k0 + 2, 0)
        wait_out(k0 + 1, 1)
        issue_loads(k0 + 3, 1)
        return carry

    lax.fori_loop(0, _K // 2 - 1, step, 0)
    kl = _K - 2
    wait_loads(kl, 0)
    compute(0)
    issue_out(kl, 0)
    wait_loads(kl + 1, 1)
    compute(1)
    issue_out(kl + 1, 1)
    wait_out(kl, 0)
    wait_out(kl + 1, 1)


@jax.jit
def kernel(inputs, positions, pe1, pe2):
    x = inputs.reshape(_N, _D)
    pos = positions.astype(jnp.int32).reshape(_N, 2)
    p0 = pos[:, 0].reshape(_NW, _K, _C)
    p1 = pos[:, 1].reshape(_NW, _K, _C)

    mesh = plsc.VectorSubcoreMesh(core_axis_name="c", subcore_axis_name="s")
    run = functools.partial(
        pl.kernel,
        out_type=jax.ShapeDtypeStruct((_N, _D), jnp.float32),
        mesh=mesh,
        scratch_types=[
            pltpu.VMEM((_K, _C), jnp.int32),
            pltpu.VMEM((_K, _C), jnp.int32),
            pltpu.VMEM((_NBUF, _C, _H), jnp.float32),
            pltpu.VMEM((_NBUF, _C, _H), jnp.float32),
            pltpu.VMEM((_NBUF, _C, _D), jnp.float32),
            pltpu.SemaphoreType.DMA((_NBUF,)),
            pltpu.SemaphoreType.DMA((_NBUF,)),
            pltpu.SemaphoreType.DMA((_NBUF,)),
            pltpu.SemaphoreType.DMA((_NBUF,)),
        ],
    )(_body)
    out = run(x, p0, p1, pe1, pe2)
    return out.reshape(_B, _L, _D)
